# baseline (speedup 1.0000x reference)
import functools

import jax
import jax.numpy as jnp
from jax import lax
from jax.experimental import pallas as pl
from jax.experimental.pallas import tpu as pltpu
from jax.experimental.pallas import tpu_sc as plsc

_EMB = 64
_WIN = 5
_HID = 512
_OUT = 50
_B = 16384

_NFLAT = _B * _WIN
_NW = 32
_PER_W = _NFLAT // _NW
_CHUNK = 128
_NCHUNK = _PER_W // _CHUNK

_sc_mesh = plsc.VectorSubcoreMesh(core_axis_name="c", subcore_axis_name="s")


def _make_sc_pair_gather(name):
    @functools.partial(
        pl.kernel,
        mesh=_sc_mesh,
        name=name,
        out_type=jax.ShapeDtypeStruct((_NFLAT, 2 * _EMB), jnp.float32),
        scratch_types=[
            pltpu.VMEM((_PER_W,), jnp.int32),
            pltpu.VMEM((2, _CHUNK, 2 * _EMB), jnp.float32),
            pltpu.SemaphoreType.DMA,
            pltpu.SemaphoreType.DMA,
        ],
    )
    def gather(table, idx, out, idx_v, rows, sem0, sem1):
        wid = lax.axis_index("s") * 2 + lax.axis_index("c")
        base = wid * _PER_W
        pltpu.sync_copy(idx.at[pl.ds(base, _PER_W)], idx_v)
        sems = (sem0, sem1)

        def start(c):
            buf = c % 2
            return pltpu.async_copy(
                table.at[idx_v.at[pl.ds(c * _CHUNK, _CHUNK)]],
                rows.at[buf], sems[buf])

        pending = start(0)
        for c in range(_NCHUNK):
            cur = pending
            if c + 1 < _NCHUNK:
                pending = start(c + 1)
            cur.wait()
            pltpu.sync_copy(rows.at[c % 2],
                            out.at[pl.ds(base + c * _CHUNK, _CHUNK)])

    return gather


_gather_w = _make_sc_pair_gather("sc_gather_word")
_gather_p = _make_sc_pair_gather("sc_gather_prefix")
_gather_s = _make_sc_pair_gather("sc_gather_suffix")


_BM = 1024
_NB = _B // _BM


def _mlp_body(*refs):
    pair = refs[0:3 * _WIN]
    par = refs[3 * _WIN:6 * _WIN]
    w1_ref, b1_ref, w2_ref, b2_ref, o_ref = refs[6 * _WIN:]
    lane = lax.broadcasted_iota(jnp.int32, (1, 2 * _EMB), 1)
    low = lane < _EMB
    pieces = []
    for w in range(_WIN):
        acc = None
        for t in range(3):
            x = pair[t * _WIN + w][...]
            p = par[t * _WIN + w][...]
            s = jnp.where(low, 1.0 - p, p)
            sel = x * s
            acc = sel if acc is None else acc + sel
        pieces.append(acc)
    y = jnp.concatenate(pieces, axis=1)
    h = jnp.tanh(
        jnp.dot(y, w1_ref[...], preferred_element_type=jnp.float32)
        + b1_ref[...])
    o_ref[...] = (
        jnp.dot(h, w2_ref[...], preferred_element_type=jnp.float32)
        + b2_ref[...])


def _win_map(w):
    return lambda i: (w * _NB + i, 0)


_mlp = pl.pallas_call(
    _mlp_body,
    grid=(_NB,),
    in_specs=(
        [pl.BlockSpec((_BM, 2 * _EMB), _win_map(w))
         for _ in range(3) for w in range(_WIN)]
        + [pl.BlockSpec((_BM, 1), _win_map(w))
           for _ in range(3) for w in range(_WIN)]
        + [
            pl.BlockSpec((_WIN * 2 * _EMB, _HID), lambda i: (0, 0)),
            pl.BlockSpec((1, _HID), lambda i: (0, 0)),
            pl.BlockSpec((_HID, _OUT), lambda i: (0, 0)),
            pl.BlockSpec((1, _OUT), lambda i: (0, 0)),
        ]
    ),
    out_specs=pl.BlockSpec((_BM, _OUT), lambda i: (i, 0)),
    out_shape=jax.ShapeDtypeStruct((_B, _OUT), jnp.float32),
)


def kernel(words, prefixes, suffixes, word_emb, prefix_emb, suffix_emb,
           W1, b1, W2, b2):
    w2 = word_emb.reshape(word_emb.shape[0] // 2, 2 * _EMB)
    p2 = prefix_emb.reshape(prefix_emb.shape[0] // 2, 2 * _EMB)
    s2 = suffix_emb.reshape(suffix_emb.shape[0] // 2, 2 * _EMB)
    iw = (words >> 1).T.reshape(-1)
    ip = (prefixes >> 1).T.reshape(-1)
    is_ = (suffixes >> 1).T.reshape(-1)
    pw = (words & 1).T.reshape(-1, 1).astype(jnp.float32)
    pp = (prefixes & 1).T.reshape(-1, 1).astype(jnp.float32)
    ps = (suffixes & 1).T.reshape(-1, 1).astype(jnp.float32)
    xw = _gather_w(w2, iw)
    xp = _gather_p(p2, ip)
    xs = _gather_s(s2, is_)
    pairs = [xw] * _WIN + [xp] * _WIN + [xs] * _WIN
    pars = [pw] * _WIN + [pp] * _WIN + [ps] * _WIN
    w1r = W1.reshape(_WIN, _EMB, _HID)
    w1dup = jnp.concatenate([w1r, w1r], axis=1).reshape(
        _WIN * 2 * _EMB, _HID)
    return _mlp(*pairs, *pars,
                w1dup, b1.reshape(1, _HID), W2, b2.reshape(1, _OUT))
